# baseline (speedup 1.0000x reference)
import functools
import math

import jax
import jax.numpy as jnp
from jax import lax
from jax.experimental import pallas as pl
from jax.experimental.pallas import tpu as pltpu
from jax.experimental.pallas import tpu_sc as plsc

THRESH2 = (2.0 * math.sin(math.radians(2.0) / 2.0)) ** 2
ALPHA = 1.0
BETA = 1.0

NC = 2
NS = 16
NW = NC * NS
LANES = 16

B = 64
ROWLEN = 128 * 1024 * 3
CHUNK = 49152
UNROLL = 8


def _make_sc_partial(b, rowlen, chunk, unroll, interpret=False):
    bpw = b // NW
    nchunk = rowlen // chunk
    groups = chunk // (3 * LANES)

    mesh = plsc.VectorSubcoreMesh(core_axis_name="c", subcore_axis_name="s",
                                  num_cores=NC, num_subcores=NS)

    @functools.partial(
        pl.kernel,
        mesh=mesh,
        out_type=jax.ShapeDtypeStruct((b, LANES), jnp.float32),
        compiler_params=pltpu.CompilerParams(needs_layout_passes=False),
        interpret=interpret,
        scratch_types=[
            pltpu.VMEM((chunk,), jnp.float32),
            pltpu.VMEM((chunk,), jnp.float32),
            pltpu.VMEM((LANES,), jnp.float32),
            pltpu.VMEM((LANES,), jnp.float32),
            pltpu.SemaphoreType.DMA,
            pltpu.SemaphoreType.DMA,
        ],
    )
    def sc_partial(z_hbm, r_hbm, out_hbm, buf0, buf1, rv, stage, sem0, sem1):
        wid = lax.axis_index("s") * NC + lax.axis_index("c")
        b0 = wid * bpw
        bufs = (buf0, buf1)
        sems = (sem0, sem1)
        base_idx = lax.iota(jnp.int32, LANES) * 3
        th = jnp.full((LANES,), THRESH2, jnp.float32)
        zero = jnp.zeros((LANES,), jnp.float32)

        copies = {}
        copies[0] = pltpu.async_copy(z_hbm.at[b0, pl.ds(0, chunk)], buf0, sem0)
        accs = []
        for lb in range(bpw):
            pltpu.sync_copy(r_hbm.at[b0 + lb], rv)
            r16 = rv[...]
            coef = jnp.full((LANES,), -0.5, jnp.float32) / (r16 * r16)
            acc = zero
            for k in range(nchunk):
                gi = lb * nchunk + k
                nxt = gi + 1
                if nxt < bpw * nchunk:
                    nlb, nk = divmod(nxt, nchunk)
                    copies[nxt] = pltpu.async_copy(
                        z_hbm.at[b0 + nlb, pl.ds(nk * chunk, chunk)],
                        bufs[nxt % 2], sems[nxt % 2])
                copies.pop(gi).wait()
                buf = bufs[gi % 2]

                def body(_, carry, buf=buf, coef=coef):
                    off, acc = carry
                    for u in range(unroll):
                        idx = base_idx + (off + u * (3 * LANES))
                        x = plsc.load_gather(buf, [idx])
                        y = plsc.load_gather(buf, [idx + 1])
                        zc = plsc.load_gather(buf, [idx + 2])
                        t = x * x + y * y + zc * zc
                        e = jnp.exp(coef * t)
                        acc = acc + jnp.where(t < th, e, zero)
                    return (off + unroll * 3 * LANES, acc)

                _, acc = lax.fori_loop(0, groups // unroll, body,
                                       (jnp.int32(0), acc))
            accs.append(acc)
        for lb in range(bpw):
            stage[...] = accs[lb]
            pltpu.sync_copy(stage, out_hbm.at[b0 + lb])

    return sc_partial


_sc_partial = _make_sc_partial(B, ROWLEN, CHUNK, UNROLL)


def _tc_finish_body(p_ref, r_ref, no_ref, raw_ref, mu_ref, s2_ref, obj_ref):
    raw = jnp.sum(p_ref[...], axis=1, keepdims=True)
    r = r_ref[...]
    a = 1.0 / (r * r)
    lam = (0.5 * THRESH2) * a
    mu_p = (1.0 - jnp.exp(-lam)) / lam
    e2 = (1.0 - jnp.exp(-2.0 * lam)) / (2.0 * lam)
    s2_p = e2 - mu_p * mu_p
    no = no_ref[...]
    mu = no * mu_p
    s2 = no * s2_p
    raw_ref[...] = raw
    mu_ref[...] = mu
    s2_ref[...] = s2
    obj_ref[...] = raw - ALPHA * mu - BETA + jnp.sqrt(s2)


def _finish(partial, R, num_obs, interpret=False):
    b = partial.shape[0]
    r2 = R.reshape(b, 1)
    no2 = jnp.reshape(num_obs, (1, 1)).astype(jnp.float32)
    outs = pl.pallas_call(
        _tc_finish_body,
        out_shape=[jax.ShapeDtypeStruct((b, 1), jnp.float32)] * 4,
        interpret=interpret,
    )(partial, r2, no2)
    raw, mu, s2, obj = (o.reshape(b) for o in outs)
    return (raw, mu, s2, obj)


def kernel(z, R, num_obs):
    zf = z.reshape(B, ROWLEN)
    r16in = jnp.broadcast_to(R.reshape(B, 1), (B, LANES))
    partial = _sc_partial(zf, r16in)
    return _finish(partial, R, num_obs)
